# baseline (device time: 18902 ns/iter reference)
import jax
import jax.numpy as jnp
from jax import lax
from jax.experimental import pallas as pl
from jax.experimental.pallas import tpu as pltpu

QROWS = 128
C = 2
R = QROWS // C


def kernel(partial, resid, gamma):
    g = gamma.reshape(1, -1)
    _, m, d = partial.shape

    def body(p_ref, r_ref, g_ref, o_ref,
             xq, rq, gv, vout, rs_buf,
             in_sems, out_sems,
             rs_send, rs_recv, a1y_send, a1y_recv, a1z_send, a1z_recv,
             a2y_send, a2y_recv, a2z_send, a2z_recv):
        my_x = lax.axis_index("x")
        my_y = lax.axis_index("y")
        my_z = lax.axis_index("z")
        xpeer = (1 - my_x, my_y, my_z)
        ypeer = (my_x, 1 - my_y, my_z)
        zpeer = (my_x, my_y, 1 - my_z)

        q = 2 * my_y + my_z
        qy = 2 * (1 - my_y) + my_z
        qz = 2 * my_y + (1 - my_z)
        qd = 2 * (1 - my_y) + (1 - my_z)

        qrows = pl.ds(q * QROWS, QROWS)
        cp_x = pltpu.make_async_copy(p_ref.at[0, qrows, :], xq, in_sems.at[0])
        cp_r = pltpu.make_async_copy(r_ref.at[qrows, :], rq, in_sems.at[1])
        cp_g = pltpu.make_async_copy(g_ref, gv, in_sems.at[2])
        cp_x.start()
        cp_r.start()
        cp_g.start()

        barrier_sem = pltpu.get_barrier_semaphore()
        for nbr in (xpeer, ypeer, zpeer):
            pl.semaphore_signal(
                barrier_sem, inc=1,
                device_id=nbr, device_id_type=pl.DeviceIdType.MESH,
            )
        pl.semaphore_wait(barrier_sem, 3)

        def rdma(src, dst, send_sem, recv_sem, peer):
            return pltpu.make_async_remote_copy(
                src_ref=src, dst_ref=dst,
                send_sem=send_sem, recv_sem=recv_sem,
                device_id=peer, device_id_type=pl.DeviceIdType.MESH,
            )

        def out_cp(quarter, chunk, sem_idx):
            rows_v = pl.ds(quarter * QROWS + chunk * R, R)
            cp = pltpu.make_async_copy(
                vout.at[rows_v, :], o_ref.at[rows_v, :], out_sems.at[sem_idx]
            )
            cp.start()
            return cp

        cp_x.wait()
        rs = []
        for c in range(C):
            rs.append(rdma(xq.at[pl.ds(c * R, R), :],
                           rs_buf.at[pl.ds(c * R, R), :],
                           rs_send.at[c], rs_recv.at[c], xpeer))
            rs[c].start()
        cp_r.wait()
        cp_g.wait()

        a1y, a1z = [], []
        out_cps = []
        for c in range(C):
            rs[c].wait_recv()
            crows = pl.ds(c * R, R)
            vrows = pl.ds(q * QROWS + c * R, R)
            yv = xq[crows, :] + rs_buf[crows, :] + rq[crows, :]
            rms = jnp.sqrt(jnp.mean(yv * yv, axis=-1, keepdims=True) + 1e-6)
            vout[vrows, :] = yv * (gv[:, :] / rms)
            a1y.append(rdma(vout.at[vrows, :], vout.at[vrows, :],
                            a1y_send.at[c], a1y_recv.at[c], ypeer))
            a1z.append(rdma(vout.at[vrows, :], vout.at[vrows, :],
                            a1z_send.at[c], a1z_recv.at[c], zpeer))
            a1y[c].start()
            a1z[c].start()
            out_cps.append(out_cp(q, c, c))

        a1z[0].wait_recv()
        rows_zf = pl.ds(qz * QROWS, R)
        a2y = rdma(vout.at[rows_zf, :], vout.at[rows_zf, :],
                   a2y_send, a2y_recv, ypeer)
        a2y.start()
        out_cps.append(out_cp(qz, 0, 2))

        a1y[1].wait_recv()
        rows_yf = pl.ds(qy * QROWS + R, R)
        a2z = rdma(vout.at[rows_yf, :], vout.at[rows_yf, :],
                   a2z_send, a2z_recv, zpeer)
        a2z.start()
        out_cps.append(out_cp(qy, 1, 3))

        a1y[0].wait_recv()
        out_cps.append(out_cp(qy, 0, 4))
        a1z[1].wait_recv()
        out_cps.append(out_cp(qz, 1, 5))
        a2y.wait_recv()
        out_cps.append(out_cp(qd, 0, 6))
        a2z.wait_recv()
        out_cps.append(out_cp(qd, 1, 7))

        for c in range(C):
            rs[c].wait_send()
            a1y[c].wait_send()
            a1z[c].wait_send()
        a2y.wait_send()
        a2z.wait_send()
        for cp in out_cps:
            cp.wait()

    return pl.pallas_call(
        body,
        out_shape=jax.ShapeDtypeStruct((m, d), jnp.float32),
        in_specs=[
            pl.BlockSpec(memory_space=pl.ANY),
            pl.BlockSpec(memory_space=pl.ANY),
            pl.BlockSpec(memory_space=pl.ANY),
        ],
        out_specs=pl.BlockSpec(memory_space=pl.ANY),
        scratch_shapes=[
            pltpu.VMEM((QROWS, d), jnp.float32),
            pltpu.VMEM((QROWS, d), jnp.float32),
            pltpu.VMEM((1, d), jnp.float32),
            pltpu.VMEM((m, d), jnp.float32),
            pltpu.VMEM((QROWS, d), jnp.float32),
            pltpu.SemaphoreType.DMA((3,)),
            pltpu.SemaphoreType.DMA((8,)),
            pltpu.SemaphoreType.DMA((C,)),
            pltpu.SemaphoreType.DMA((C,)),
            pltpu.SemaphoreType.DMA((C,)),
            pltpu.SemaphoreType.DMA((C,)),
            pltpu.SemaphoreType.DMA((C,)),
            pltpu.SemaphoreType.DMA((C,)),
            pltpu.SemaphoreType.DMA,
            pltpu.SemaphoreType.DMA,
            pltpu.SemaphoreType.DMA,
            pltpu.SemaphoreType.DMA,
        ],
        compiler_params=pltpu.CompilerParams(collective_id=0),
    )(partial, resid, g)


# device time: 17784 ns/iter; 1.0629x vs baseline; 1.0629x over previous
import jax
import jax.numpy as jnp
from jax import lax
from jax.experimental import pallas as pl
from jax.experimental.pallas import tpu as pltpu

QROWS = 128
C = 8
R = QROWS // C


def kernel(partial, resid, gamma):
    _, m, d = partial.shape

    def body(p_ref, r_ref, g_ref, o_ref,
             xq, rq, gv, vout, rs_buf,
             in_sems, out_sems,
             rs_send, rs_recv, a1y_send, a1y_recv, a1z_send, a1z_recv,
             a2y_send, a2y_recv, a2z_send, a2z_recv):
        my_x = lax.axis_index("x")
        my_y = lax.axis_index("y")
        my_z = lax.axis_index("z")
        xpeer = (1 - my_x, my_y, my_z)
        ypeer = (my_x, 1 - my_y, my_z)
        zpeer = (my_x, my_y, 1 - my_z)

        q = 2 * my_y + my_z
        qy = 2 * (1 - my_y) + my_z
        qz = 2 * my_y + (1 - my_z)
        qd = 2 * (1 - my_y) + (1 - my_z)

        qrows = pl.ds(q * QROWS, QROWS)
        cp_x = []
        for c in range(C):
            cp_x.append(pltpu.make_async_copy(
                p_ref.at[0, pl.ds(q * QROWS + c * R, R), :],
                xq.at[pl.ds(c * R, R), :], in_sems.at[c]))
            cp_x[c].start()
        cp_r = pltpu.make_async_copy(r_ref.at[qrows, :], rq, in_sems.at[C])
        cp_g = pltpu.make_async_copy(g_ref, gv.at[0, :], in_sems.at[C + 1])
        cp_r.start()
        cp_g.start()

        barrier_sem = pltpu.get_barrier_semaphore()
        for nbr in (xpeer, ypeer, zpeer):
            pl.semaphore_signal(
                barrier_sem, inc=1,
                device_id=nbr, device_id_type=pl.DeviceIdType.MESH,
            )
        pl.semaphore_wait(barrier_sem, 3)

        def rdma(src, dst, send_sem, recv_sem, peer):
            return pltpu.make_async_remote_copy(
                src_ref=src, dst_ref=dst,
                send_sem=send_sem, recv_sem=recv_sem,
                device_id=peer, device_id_type=pl.DeviceIdType.MESH,
            )

        out_cps = []

        def out_cp(quarter, chunk):
            rows_v = pl.ds(quarter * QROWS + chunk * R, R)
            cp = pltpu.make_async_copy(
                vout.at[rows_v, :], o_ref.at[rows_v, :],
                out_sems.at[len(out_cps)],
            )
            cp.start()
            out_cps.append(cp)

        rs = []
        for c in range(C):
            cp_x[c].wait()
            rs.append(rdma(xq.at[pl.ds(c * R, R), :],
                           rs_buf.at[pl.ds(c * R, R), :],
                           rs_send.at[c], rs_recv.at[c], xpeer))
            rs[c].start()
        cp_r.wait()
        cp_g.wait()

        a1y, a1z = [], []
        for c in range(C):
            rs[c].wait_recv()
            crows = pl.ds(c * R, R)
            vrows = pl.ds(q * QROWS + c * R, R)
            yv = xq[crows, :] + rs_buf[crows, :] + rq[crows, :]
            rms = jnp.sqrt(jnp.mean(yv * yv, axis=-1, keepdims=True) + 1e-6)
            vout[vrows, :] = yv * (gv[:, :] / rms)
            a1y.append(rdma(vout.at[vrows, :], vout.at[vrows, :],
                            a1y_send.at[c], a1y_recv.at[c], ypeer))
            a1z.append(rdma(vout.at[vrows, :], vout.at[vrows, :],
                            a1z_send.at[c], a1z_recv.at[c], zpeer))
            a1y[c].start()
            a1z[c].start()
            out_cp(q, c)

        a2 = []
        for c in range(C):
            if c % 2 == 0:
                a1z[c].wait_recv()
                out_cp(qz, c)
                fwd = rdma(vout.at[pl.ds(qz * QROWS + c * R, R), :],
                           vout.at[pl.ds(qz * QROWS + c * R, R), :],
                           a2y_send.at[c // 2], a2y_recv.at[c // 2], ypeer)
            else:
                a1y[c].wait_recv()
                out_cp(qy, c)
                fwd = rdma(vout.at[pl.ds(qy * QROWS + c * R, R), :],
                           vout.at[pl.ds(qy * QROWS + c * R, R), :],
                           a2z_send.at[c // 2], a2z_recv.at[c // 2], zpeer)
            fwd.start()
            a2.append(fwd)

        for c in range(C):
            if c % 2 == 0:
                a1y[c].wait_recv()
                out_cp(qy, c)
            else:
                a1z[c].wait_recv()
                out_cp(qz, c)

        for c in range(C):
            a2[c].wait_recv()
            out_cp(qd, c)

        for c in range(C):
            rs[c].wait_send()
            a1y[c].wait_send()
            a1z[c].wait_send()
            a2[c].wait_send()
        for cp in out_cps:
            cp.wait()

    return pl.pallas_call(
        body,
        out_shape=jax.ShapeDtypeStruct((m, d), jnp.float32),
        in_specs=[
            pl.BlockSpec(memory_space=pl.ANY),
            pl.BlockSpec(memory_space=pl.ANY),
            pl.BlockSpec(memory_space=pl.ANY),
        ],
        out_specs=pl.BlockSpec(memory_space=pl.ANY),
        scratch_shapes=[
            pltpu.VMEM((QROWS, d), jnp.float32),
            pltpu.VMEM((QROWS, d), jnp.float32),
            pltpu.VMEM((1, d), jnp.float32),
            pltpu.VMEM((m, d), jnp.float32),
            pltpu.VMEM((QROWS, d), jnp.float32),
            pltpu.SemaphoreType.DMA((C + 2,)),
            pltpu.SemaphoreType.DMA((4 * C,)),
            pltpu.SemaphoreType.DMA((C,)),
            pltpu.SemaphoreType.DMA((C,)),
            pltpu.SemaphoreType.DMA((C,)),
            pltpu.SemaphoreType.DMA((C,)),
            pltpu.SemaphoreType.DMA((C,)),
            pltpu.SemaphoreType.DMA((C,)),
            pltpu.SemaphoreType.DMA(((C + 1) // 2,)),
            pltpu.SemaphoreType.DMA(((C + 1) // 2,)),
            pltpu.SemaphoreType.DMA((C // 2,)),
            pltpu.SemaphoreType.DMA((C // 2,)),
        ],
        compiler_params=pltpu.CompilerParams(collective_id=0),
    )(partial, resid, gamma)
